# baseline (speedup 1.0000x reference)
import jax
import jax.numpy as jnp
from jax import lax
from jax.experimental import pallas as pl
from jax.experimental.pallas import tpu as pltpu
from jax.experimental.pallas import tpu_sc as plsc

N0 = 10000
E0 = 160000
NP = 10240
NC, NS = 2, 16
NW = NC * NS
EPW = E0 // NW
CH0, CH1 = 2560, 2440
RPS = NP // NS
EQ = E0 // 4
PBS = 800
NBS = 1024


def _mesh():
    return plsc.VectorSubcoreMesh(core_axis_name="c", subcore_axis_name="s",
                                  num_cores=NC, num_subcores=NS)


def _sc_params():
    return pltpu.CompilerParams(use_tc_tiling_on_sc=False)


def _sc_gather(table, idx, d, dt):
    def body(tab_ref, idx_ref, out_ref, idx_v0, idx_v1, rows_v, sem):
        wid = lax.axis_index("s") * NC + lax.axis_index("c")
        base = wid * EPW
        for off, ch, idx_v in ((base, CH0, idx_v0), (base + CH0, CH1, idx_v1)):
            pltpu.sync_copy(idx_ref.at[pl.ds(off, ch)], idx_v)
            pltpu.async_copy(tab_ref.at[idx_v], rows_v.at[pl.ds(0, ch)], sem).wait()
            pltpu.sync_copy(rows_v.at[pl.ds(0, ch)], out_ref.at[pl.ds(off, ch)])

    return pl.kernel(
        body,
        out_type=jax.ShapeDtypeStruct((E0, d), dt),
        mesh=_mesh(),
        scratch_types=[pltpu.VMEM((CH0,), jnp.int32),
                       pltpu.VMEM((CH1,), jnp.int32),
                       pltpu.VMEM((CH0, d), dt),
                       pltpu.SemaphoreType.DMA],
        compiler_params=_sc_params(),
    )(table, idx)


def _sc_scatter(msg, dstidx, zrows, d):
    def body(msg_ref, dst_ref, z_ref, out_ref, idx_v0, idx_v1, rows_v, acc):
        c = lax.axis_index("c")
        s = lax.axis_index("s")
        wid = s * NC + c
        r0 = s * RPS
        pltpu.sync_copy(z_ref.at[pl.ds(r0, RPS)], acc.at[pl.ds(r0, RPS)])
        plsc.subcore_barrier()
        base = wid * EPW
        for off, ch, idx_v in ((base, CH0, idx_v0), (base + CH0, CH1, idx_v1)):
            pltpu.sync_copy(dst_ref.at[pl.ds(off, ch)], idx_v)
            pltpu.sync_copy(msg_ref.at[pl.ds(off, ch)], rows_v.at[pl.ds(0, ch)])
            pltpu.sync_copy(rows_v.at[pl.ds(0, ch)], acc.at[idx_v], add=True)
        plsc.subcore_barrier()
        pltpu.sync_copy(acc.at[pl.ds(r0, RPS)], out_ref.at[c, pl.ds(r0, RPS)])

    return pl.kernel(
        body,
        out_type=jax.ShapeDtypeStruct((NC, NP, d), jnp.float32),
        mesh=_mesh(),
        scratch_types=[pltpu.VMEM((CH0,), jnp.int32),
                       pltpu.VMEM((CH1,), jnp.int32),
                       pltpu.VMEM((CH0, d), jnp.float32),
                       pltpu.VMEM_SHARED((NP, d), jnp.float32)],
        compiler_params=_sc_params(),
    )(msg, dstidx, zrows)


def _sc_gcn(g, srcidx, dstidx, zrows, d):
    def body(g_ref, src_ref, dst_ref, z_ref, out_ref,
             si_v0, si_v1, di_v0, di_v1, rows_v, acc, sem):
        c = lax.axis_index("c")
        s = lax.axis_index("s")
        wid = s * NC + c
        r0 = s * RPS
        pltpu.sync_copy(z_ref.at[pl.ds(r0, RPS)], acc.at[pl.ds(r0, RPS)])
        plsc.subcore_barrier()
        base = wid * EPW
        for off, ch, si_v, di_v in ((base, CH0, si_v0, di_v0),
                                    (base + CH0, CH1, si_v1, di_v1)):
            pltpu.sync_copy(src_ref.at[pl.ds(off, ch)], si_v)
            pltpu.sync_copy(dst_ref.at[pl.ds(off, ch)], di_v)
            pltpu.async_copy(g_ref.at[si_v], rows_v.at[pl.ds(0, ch)], sem).wait()
            pltpu.sync_copy(rows_v.at[pl.ds(0, ch)], acc.at[di_v], add=True)
        plsc.subcore_barrier()
        pltpu.sync_copy(acc.at[pl.ds(r0, RPS)], out_ref.at[c, pl.ds(r0, RPS)])

    return pl.kernel(
        body,
        out_type=jax.ShapeDtypeStruct((NC, NP, d), jnp.float32),
        mesh=_mesh(),
        scratch_types=[pltpu.VMEM((CH0,), jnp.int32),
                       pltpu.VMEM((CH1,), jnp.int32),
                       pltpu.VMEM((CH0,), jnp.int32),
                       pltpu.VMEM((CH1,), jnp.int32),
                       pltpu.VMEM((CH0, d), jnp.float32),
                       pltpu.VMEM_SHARED((NP, d), jnp.float32),
                       pltpu.SemaphoreType.DMA],
        compiler_params=_sc_params(),
    )(g, srcidx, dstidx, zrows)


def _tc_edge_msg(xjp, eap, ra, rb, w, bem, deg_col):
    def body(xjp_ref, eap_ref, ra_ref, rb_ref, w_ref, bem_ref, out_ref):
        xjb = xjp_ref[...].astype(jnp.bfloat16)
        eab = eap_ref[...]
        a = jnp.dot(xjb, ra_ref[...],
                    preferred_element_type=jnp.float32).astype(jnp.bfloat16)
        b = jnp.dot(eab, rb_ref[...],
                    preferred_element_type=jnp.float32).astype(jnp.bfloat16)
        m = jnp.dot(a * b, w_ref[...], preferred_element_type=jnp.float32)
        m = m + jnp.dot(xjb, bem_ref[...], preferred_element_type=jnp.float32)
        if deg_col is not None:
            lanes = lax.broadcasted_iota(jnp.int32, (PBS, 128), 1)
            m = m + jnp.where(lanes % 32 == deg_col, 1.0, 0.0)
        out_ref[...] = m

    return pl.pallas_call(
        body,
        grid=(EQ // PBS,),
        in_specs=[pl.BlockSpec((PBS, 128), lambda i: (i, 0)),
                  pl.BlockSpec((PBS, 64), lambda i: (i, 0)),
                  pl.BlockSpec((128, 2048), lambda i: (0, 0)),
                  pl.BlockSpec((64, 2048), lambda i: (0, 0)),
                  pl.BlockSpec((2048, 128), lambda i: (0, 0)),
                  pl.BlockSpec((128, 128), lambda i: (0, 0))],
        out_specs=pl.BlockSpec((PBS, 128), lambda i: (i, 0)),
        out_shape=jax.ShapeDtypeStruct((EQ, 128), jnp.float32),
    )(xjp, eap, ra, rb, w, bem)


def _tc_h1(a0, a1, xp, r1p, bi1):
    def body(a0_ref, a1_ref, x_ref, r_ref, b_ref, h_ref, dinv_ref):
        ssum = a0_ref[...] + a1_ref[...]
        hf = ssum + jnp.dot(x_ref[...], r_ref[...],
                            preferred_element_type=jnp.float32) + b_ref[...]
        lanes = lax.broadcasted_iota(jnp.int32, (NBS, 32), 1)
        h_ref[...] = jnp.where(lanes < 24, jnp.maximum(hf, 0.0), 0.0)
        deg = ssum[:, 24:25] + 1.0
        dinv_ref[...] = jnp.broadcast_to(lax.rsqrt(deg), (NBS, 8))

    return pl.pallas_call(
        body,
        grid=(NP // NBS,),
        in_specs=[pl.BlockSpec((NBS, 32), lambda i: (i, 0)),
                  pl.BlockSpec((NBS, 32), lambda i: (i, 0)),
                  pl.BlockSpec((NBS, 32), lambda i: (i, 0)),
                  pl.BlockSpec((32, 32), lambda i: (0, 0)),
                  pl.BlockSpec((1, 32), lambda i: (0, 0))],
        out_specs=[pl.BlockSpec((NBS, 32), lambda i: (i, 0)),
                   pl.BlockSpec((NBS, 8), lambda i: (i, 0))],
        out_shape=[jax.ShapeDtypeStruct((NP, 32), jnp.float32),
                   jax.ShapeDtypeStruct((NP, 8), jnp.float32)],
    )(a0, a1, xp, r1p, bi1)


def _tc_g(a0, a1, h1, dinv, r2p, bi2, wc):
    def body(a0_ref, a1_ref, h1_ref, dv_ref, r_ref, b_ref, wc_ref, g_ref):
        h2 = a0_ref[...] + a1_ref[...] + jnp.dot(
            h1_ref[...], r_ref[...], preferred_element_type=jnp.float32) + b_ref[...]
        h2 = jnp.maximum(h2, 0.0)
        g_ref[...] = dv_ref[:, 0:1] * jnp.dot(
            h2, wc_ref[...], preferred_element_type=jnp.float32)

    return pl.pallas_call(
        body,
        grid=(NP // NBS,),
        in_specs=[pl.BlockSpec((NBS, 32), lambda i: (i, 0)),
                  pl.BlockSpec((NBS, 32), lambda i: (i, 0)),
                  pl.BlockSpec((NBS, 32), lambda i: (i, 0)),
                  pl.BlockSpec((NBS, 8), lambda i: (i, 0)),
                  pl.BlockSpec((32, 32), lambda i: (0, 0)),
                  pl.BlockSpec((1, 32), lambda i: (0, 0)),
                  pl.BlockSpec((32, 16), lambda i: (0, 0))],
        out_specs=pl.BlockSpec((NBS, 16), lambda i: (i, 0)),
        out_shape=jax.ShapeDtypeStruct((NP, 16), jnp.float32),
    )(a0, a1, h1, dinv, r2p, bi2, wc)


def _tc_final(s0, s1, g, dinv, bc):
    def body(s0_ref, s1_ref, g_ref, dv_ref, b_ref, o_ref):
        o_ref[...] = dv_ref[:, 0:1] * (s0_ref[...] + s1_ref[...] + g_ref[...]) + b_ref[...]

    return pl.pallas_call(
        body,
        grid=(NP // NBS,),
        in_specs=[pl.BlockSpec((NBS, 16), lambda i: (i, 0)),
                  pl.BlockSpec((NBS, 16), lambda i: (i, 0)),
                  pl.BlockSpec((NBS, 16), lambda i: (i, 0)),
                  pl.BlockSpec((NBS, 8), lambda i: (i, 0)),
                  pl.BlockSpec((1, 16), lambda i: (0, 0))],
        out_specs=pl.BlockSpec((NBS, 16), lambda i: (i, 0)),
        out_shape=jax.ShapeDtypeStruct((NP, 16), jnp.float32),
    )(s0, s1, g, dinv, bc)


def kernel(x, edge_index, edge_attr, We1, be1, root1, bias1,
           We2, be2, root2, bias2, Wmu, bmu, Wls, bls):
    f32 = jnp.float32
    src = edge_index[0]
    dst = edge_index[1]
    ea = edge_attr
    xp = jnp.zeros((NP, 32), f32).at[:N0].set(x)

    bf16 = jnp.bfloat16
    colr = jnp.arange(2048)
    s_c, rem = colr // 512, colr % 512
    i_c, f_c = rem // 16, rem % 16
    rowa = jnp.arange(128)
    ra = ((rowa[:, None] // 32 == s_c[None, :])
          & (rowa[:, None] % 32 == i_c[None, :])).astype(bf16)
    rowb = jnp.arange(64)
    rb = ((rowb[:, None] // 16 == s_c[None, :])
          & (rowb[:, None] % 16 == f_c[None, :])).astype(bf16)

    eye4 = jnp.eye(4, dtype=f32)
    w1 = jnp.kron(eye4, jnp.zeros((512, 32), f32).at[:, :24].set(
        We1.reshape(16, 32, 24).transpose(1, 0, 2).reshape(512, 24))).astype(bf16)
    b1m = jnp.kron(eye4, jnp.zeros((32, 32), f32).at[:, :24].set(
        be1.reshape(32, 24))).astype(bf16)
    w2 = jnp.kron(eye4, jnp.zeros((512, 32), f32).at[:384, :16].set(
        We2.reshape(16, 24, 16).transpose(1, 0, 2).reshape(384, 16))).astype(bf16)
    b2m = jnp.kron(eye4, jnp.zeros((32, 32), f32).at[:24, :16].set(
        be2.reshape(24, 16))).astype(bf16)
    r1p = jnp.zeros((32, 32), f32).at[:, :24].set(root1)
    bi1 = jnp.zeros((1, 32), f32).at[0, :24].set(bias1)
    r2p = jnp.zeros((32, 32), f32).at[:24, :16].set(root2)
    bi2 = jnp.zeros((1, 32), f32).at[0, :16].set(bias2)
    wc = jnp.zeros((32, 16), f32).at[:16].set(jnp.concatenate([Wmu, Wls], axis=1))
    bc = jnp.concatenate([bmu, bls]).reshape(1, 16)
    z32 = jnp.zeros((NP, 32), f32)
    z16 = jnp.zeros((NP, 16), f32)

    eap = ea.astype(bf16).reshape(EQ, 64)

    xj = _sc_gather(xp, src, 32, f32)
    msg1p = _tc_edge_msg(xj.reshape(EQ, 128), eap, ra, rb, w1, b1m, deg_col=24)
    acc1 = _sc_scatter(msg1p.reshape(E0, 32), dst, z32, 32)
    h1, dinv = _tc_h1(acc1[0], acc1[1], xp, r1p, bi1)

    h1j = _sc_gather(h1, src, 32, f32)
    msg2p = _tc_edge_msg(h1j.reshape(EQ, 128), eap, ra, rb, w2, b2m, deg_col=None)
    acc2 = _sc_scatter(msg2p.reshape(E0, 32), dst, z32, 32)
    g = _tc_g(acc2[0], acc2[1], h1, dinv, r2p, bi2, wc)

    s = _sc_gcn(g, src, dst, z16, 16)
    o = _tc_final(s[0], s[1], g, dinv, bc)
    return (o[:N0, :8], o[:N0, 8:16])


# speedup vs baseline: 1.0615x; 1.0615x over previous
import jax
import jax.numpy as jnp
from jax import lax
from jax.experimental import pallas as pl
from jax.experimental.pallas import tpu as pltpu
from jax.experimental.pallas import tpu_sc as plsc

N0 = 10000
E0 = 160000
NP = 10240
NC, NS = 2, 16
NW = NC * NS
EPW = E0 // NW
CH0, CH1 = 2560, 2440
RPS = NP // NS
EQ = E0 // 4
PBS = 800
NBS = 1024


def _mesh():
    return plsc.VectorSubcoreMesh(core_axis_name="c", subcore_axis_name="s",
                                  num_cores=NC, num_subcores=NS)


def _sc_params():
    return pltpu.CompilerParams(use_tc_tiling_on_sc=False)


def _sc_gather(table, idx, d, dt):
    def body(tab_ref, idx_ref, out_ref, idx_v0, idx_v1, rows_v, sem):
        wid = lax.axis_index("s") * NC + lax.axis_index("c")
        base = wid * EPW
        for off, ch, idx_v in ((base, CH0, idx_v0), (base + CH0, CH1, idx_v1)):
            pltpu.sync_copy(idx_ref.at[pl.ds(off, ch)], idx_v)
            pltpu.async_copy(tab_ref.at[idx_v], rows_v.at[pl.ds(0, ch)], sem).wait()
            pltpu.sync_copy(rows_v.at[pl.ds(0, ch)], out_ref.at[pl.ds(off, ch)])

    return pl.kernel(
        body,
        out_type=jax.ShapeDtypeStruct((E0, d), dt),
        mesh=_mesh(),
        scratch_types=[pltpu.VMEM((CH0,), jnp.int32),
                       pltpu.VMEM((CH1,), jnp.int32),
                       pltpu.VMEM((CH0, d), dt),
                       pltpu.SemaphoreType.DMA],
        compiler_params=_sc_params(),
    )(table, idx)


def _sc_scatter(msg, dstidx, zrows, d):
    def body(msg_ref, dst_ref, z_ref, out_ref, idx_v0, idx_v1, rows_v, acc):
        c = lax.axis_index("c")
        s = lax.axis_index("s")
        wid = s * NC + c
        r0 = s * RPS
        pltpu.sync_copy(z_ref.at[pl.ds(r0, RPS)], acc.at[pl.ds(r0, RPS)])
        plsc.subcore_barrier()
        base = wid * EPW
        for off, ch, idx_v in ((base, CH0, idx_v0), (base + CH0, CH1, idx_v1)):
            pltpu.sync_copy(dst_ref.at[pl.ds(off, ch)], idx_v)
            pltpu.sync_copy(msg_ref.at[pl.ds(off, ch)], rows_v.at[pl.ds(0, ch)])
            pltpu.sync_copy(rows_v.at[pl.ds(0, ch)], acc.at[idx_v], add=True)
        plsc.subcore_barrier()
        pltpu.sync_copy(acc.at[pl.ds(r0, RPS)], out_ref.at[c, pl.ds(r0, RPS)])

    return pl.kernel(
        body,
        out_type=jax.ShapeDtypeStruct((NC, NP, d), jnp.float32),
        mesh=_mesh(),
        scratch_types=[pltpu.VMEM((CH0,), jnp.int32),
                       pltpu.VMEM((CH1,), jnp.int32),
                       pltpu.VMEM((CH0, d), jnp.float32),
                       pltpu.VMEM_SHARED((NP, d), jnp.float32)],
        compiler_params=_sc_params(),
    )(msg, dstidx, zrows)


def _sc_gcn(g, srcidx, dstidx, zrows, d):
    def body(g_ref, src_ref, dst_ref, z_ref, out_ref,
             si_v0, si_v1, di_v0, di_v1, rows_v, acc, sem):
        c = lax.axis_index("c")
        s = lax.axis_index("s")
        wid = s * NC + c
        r0 = s * RPS
        pltpu.sync_copy(z_ref.at[pl.ds(r0, RPS)], acc.at[pl.ds(r0, RPS)])
        plsc.subcore_barrier()
        base = wid * EPW
        for off, ch, si_v, di_v in ((base, CH0, si_v0, di_v0),
                                    (base + CH0, CH1, si_v1, di_v1)):
            pltpu.sync_copy(src_ref.at[pl.ds(off, ch)], si_v)
            pltpu.sync_copy(dst_ref.at[pl.ds(off, ch)], di_v)
            pltpu.async_copy(g_ref.at[si_v], rows_v.at[pl.ds(0, ch)], sem).wait()
            pltpu.sync_copy(rows_v.at[pl.ds(0, ch)], acc.at[di_v], add=True)
        plsc.subcore_barrier()
        pltpu.sync_copy(acc.at[pl.ds(r0, RPS)], out_ref.at[c, pl.ds(r0, RPS)])

    return pl.kernel(
        body,
        out_type=jax.ShapeDtypeStruct((NC, NP, d), jnp.float32),
        mesh=_mesh(),
        scratch_types=[pltpu.VMEM((CH0,), jnp.int32),
                       pltpu.VMEM((CH1,), jnp.int32),
                       pltpu.VMEM((CH0,), jnp.int32),
                       pltpu.VMEM((CH1,), jnp.int32),
                       pltpu.VMEM((CH0, d), jnp.float32),
                       pltpu.VMEM_SHARED((NP, d), jnp.float32),
                       pltpu.SemaphoreType.DMA],
        compiler_params=_sc_params(),
    )(g, srcidx, dstidx, zrows)


def _tc_edge_msg(xjp, eap, ra, rb, w, bem, deg_col):
    def body(xjp_ref, eap_ref, ra_ref, rb_ref, w_ref, bem_ref, out_ref):
        xjb = xjp_ref[...].astype(jnp.bfloat16)
        eab = eap_ref[...].astype(jnp.bfloat16)
        a = jnp.dot(xjb, ra_ref[...],
                    preferred_element_type=jnp.float32).astype(jnp.bfloat16)
        b = jnp.dot(eab, rb_ref[...],
                    preferred_element_type=jnp.float32).astype(jnp.bfloat16)
        m = jnp.dot(a * b, w_ref[...], preferred_element_type=jnp.float32)
        m = m + jnp.dot(xjb, bem_ref[...], preferred_element_type=jnp.float32)
        if deg_col is not None:
            lanes = lax.broadcasted_iota(jnp.int32, (PBS, 128), 1)
            m = m + jnp.where(lanes % 32 == deg_col, 1.0, 0.0)
        out_ref[...] = m

    return pl.pallas_call(
        body,
        grid=(EQ // PBS,),
        in_specs=[pl.BlockSpec((PBS, 128), lambda i: (i, 0)),
                  pl.BlockSpec((PBS, 64), lambda i: (i, 0)),
                  pl.BlockSpec((128, 2048), lambda i: (0, 0)),
                  pl.BlockSpec((64, 2048), lambda i: (0, 0)),
                  pl.BlockSpec((2048, 128), lambda i: (0, 0)),
                  pl.BlockSpec((128, 128), lambda i: (0, 0))],
        out_specs=pl.BlockSpec((PBS, 128), lambda i: (i, 0)),
        out_shape=jax.ShapeDtypeStruct((EQ, 128), jnp.float32),
    )(xjp, eap, ra, rb, w, bem)


def _tc_h1(a0, a1, xp, r1p, bi1):
    def body(a0_ref, a1_ref, x_ref, r_ref, b_ref, h_ref, dinv_ref):
        ssum = a0_ref[0] + a1_ref[0]
        hf = ssum + jnp.dot(x_ref[...], r_ref[...],
                            preferred_element_type=jnp.float32) + b_ref[...]
        lanes = lax.broadcasted_iota(jnp.int32, (NBS, 32), 1)
        h_ref[...] = jnp.where(lanes < 24, jnp.maximum(hf, 0.0), 0.0)
        deg = ssum[:, 24:25] + 1.0
        dinv_ref[...] = jnp.broadcast_to(lax.rsqrt(deg), (NBS, 8))

    return pl.pallas_call(
        body,
        grid=(NP // NBS,),
        in_specs=[pl.BlockSpec((1, NBS, 32), lambda i: (0, i, 0)),
                  pl.BlockSpec((1, NBS, 32), lambda i: (1, i, 0)),
                  pl.BlockSpec((NBS, 32), lambda i: (i, 0)),
                  pl.BlockSpec((32, 32), lambda i: (0, 0)),
                  pl.BlockSpec((1, 32), lambda i: (0, 0))],
        out_specs=[pl.BlockSpec((NBS, 32), lambda i: (i, 0)),
                   pl.BlockSpec((NBS, 8), lambda i: (i, 0))],
        out_shape=[jax.ShapeDtypeStruct((NP, 32), jnp.float32),
                   jax.ShapeDtypeStruct((NP, 8), jnp.float32)],
    )(a0, a1, xp, r1p, bi1)


def _tc_g(a0, a1, h1, dinv, r2p, bi2, wc):
    def body(a0_ref, a1_ref, h1_ref, dv_ref, r_ref, b_ref, wc_ref, g_ref):
        h2 = a0_ref[0] + a1_ref[0] + jnp.dot(
            h1_ref[...], r_ref[...], preferred_element_type=jnp.float32) + b_ref[...]
        h2 = jnp.maximum(h2, 0.0)
        g_ref[...] = dv_ref[:, 0:1] * jnp.dot(
            h2, wc_ref[...], preferred_element_type=jnp.float32)

    return pl.pallas_call(
        body,
        grid=(NP // NBS,),
        in_specs=[pl.BlockSpec((1, NBS, 32), lambda i: (0, i, 0)),
                  pl.BlockSpec((1, NBS, 32), lambda i: (1, i, 0)),
                  pl.BlockSpec((NBS, 32), lambda i: (i, 0)),
                  pl.BlockSpec((NBS, 8), lambda i: (i, 0)),
                  pl.BlockSpec((32, 32), lambda i: (0, 0)),
                  pl.BlockSpec((1, 32), lambda i: (0, 0)),
                  pl.BlockSpec((32, 16), lambda i: (0, 0))],
        out_specs=pl.BlockSpec((NBS, 16), lambda i: (i, 0)),
        out_shape=jax.ShapeDtypeStruct((NP, 16), jnp.float32),
    )(a0, a1, h1, dinv, r2p, bi2, wc)


def _tc_final(s0, s1, g, dinv, bc):
    def body(s0_ref, s1_ref, g_ref, dv_ref, b_ref, o_ref):
        o_ref[...] = dv_ref[:, 0:1] * (s0_ref[0] + s1_ref[0] + g_ref[...]) + b_ref[...]

    return pl.pallas_call(
        body,
        grid=(NP // NBS,),
        in_specs=[pl.BlockSpec((1, NBS, 16), lambda i: (0, i, 0)),
                  pl.BlockSpec((1, NBS, 16), lambda i: (1, i, 0)),
                  pl.BlockSpec((NBS, 16), lambda i: (i, 0)),
                  pl.BlockSpec((NBS, 8), lambda i: (i, 0)),
                  pl.BlockSpec((1, 16), lambda i: (0, 0))],
        out_specs=pl.BlockSpec((NBS, 16), lambda i: (i, 0)),
        out_shape=jax.ShapeDtypeStruct((NP, 16), jnp.float32),
    )(s0, s1, g, dinv, bc)


def kernel(x, edge_index, edge_attr, We1, be1, root1, bias1,
           We2, be2, root2, bias2, Wmu, bmu, Wls, bls):
    f32 = jnp.float32
    src = edge_index[0]
    dst = edge_index[1]
    ea = edge_attr
    xp = jnp.zeros((NP, 32), f32).at[:N0].set(x)

    bf16 = jnp.bfloat16
    colr = jnp.arange(2048)
    s_c, rem = colr // 512, colr % 512
    i_c, f_c = rem // 16, rem % 16
    rowa = jnp.arange(128)
    ra = ((rowa[:, None] // 32 == s_c[None, :])
          & (rowa[:, None] % 32 == i_c[None, :])).astype(bf16)
    rowb = jnp.arange(64)
    rb = ((rowb[:, None] // 16 == s_c[None, :])
          & (rowb[:, None] % 16 == f_c[None, :])).astype(bf16)

    eye4 = jnp.eye(4, dtype=f32)
    w1 = jnp.kron(eye4, jnp.zeros((512, 32), f32).at[:, :24].set(
        We1.reshape(16, 32, 24).transpose(1, 0, 2).reshape(512, 24))).astype(bf16)
    b1m = jnp.kron(eye4, jnp.zeros((32, 32), f32).at[:, :24].set(
        be1.reshape(32, 24))).astype(bf16)
    w2 = jnp.kron(eye4, jnp.zeros((512, 32), f32).at[:384, :16].set(
        We2.reshape(16, 24, 16).transpose(1, 0, 2).reshape(384, 16))).astype(bf16)
    b2m = jnp.kron(eye4, jnp.zeros((32, 32), f32).at[:24, :16].set(
        be2.reshape(24, 16))).astype(bf16)
    r1p = jnp.zeros((32, 32), f32).at[:, :24].set(root1)
    bi1 = jnp.zeros((1, 32), f32).at[0, :24].set(bias1)
    r2p = jnp.zeros((32, 32), f32).at[:24, :16].set(root2)
    bi2 = jnp.zeros((1, 32), f32).at[0, :16].set(bias2)
    wc = jnp.zeros((32, 16), f32).at[:16].set(jnp.concatenate([Wmu, Wls], axis=1))
    bc = jnp.concatenate([bmu, bls]).reshape(1, 16)
    z32 = jnp.zeros((NP, 32), f32)
    z16 = jnp.zeros((NP, 16), f32)

    eap = ea.reshape(EQ, 64)

    xj = _sc_gather(xp, src, 32, f32)
    msg1p = _tc_edge_msg(xj.reshape(EQ, 128), eap, ra, rb, w1, b1m, deg_col=24)
    acc1 = _sc_scatter(msg1p.reshape(E0, 32), dst, z32, 32)
    h1, dinv = _tc_h1(acc1, acc1, xp, r1p, bi1)

    h1j = _sc_gather(h1, src, 32, f32)
    msg2p = _tc_edge_msg(h1j.reshape(EQ, 128), eap, ra, rb, w2, b2m, deg_col=None)
    acc2 = _sc_scatter(msg2p.reshape(E0, 32), dst, z32, 32)
    g = _tc_g(acc2, acc2, h1, dinv, r2p, bi2, wc)

    s = _sc_gcn(g, src, dst, z16, 16)
    o = _tc_final(s, s, g, dinv, bc)
    return (o[:N0, :8], o[:N0, 8:16])


# speedup vs baseline: 1.0845x; 1.0217x over previous
import jax
import jax.numpy as jnp
from jax import lax
from jax.experimental import pallas as pl
from jax.experimental.pallas import tpu as pltpu
from jax.experimental.pallas import tpu_sc as plsc

N0 = 10000
E0 = 160000
NP = 10240
NC, NS = 2, 16
NW = NC * NS
EPW = E0 // NW
CH0, CH1 = 2560, 2440
RPS = NP // NS
EQ = E0 // 4
PBS = 1600
NBS = 1024


def _mesh():
    return plsc.VectorSubcoreMesh(core_axis_name="c", subcore_axis_name="s",
                                  num_cores=NC, num_subcores=NS)


def _sc_params():
    return pltpu.CompilerParams(use_tc_tiling_on_sc=False)


def _sc_gather(table, idx, d, dt):
    def body(tab_ref, idx_ref, out_ref, idx_v0, idx_v1, rows_v, sem):
        wid = lax.axis_index("s") * NC + lax.axis_index("c")
        base = wid * EPW
        for off, ch, idx_v in ((base, CH0, idx_v0), (base + CH0, CH1, idx_v1)):
            pltpu.sync_copy(idx_ref.at[pl.ds(off, ch)], idx_v)
            pltpu.async_copy(tab_ref.at[idx_v], rows_v.at[pl.ds(0, ch)], sem).wait()
            pltpu.sync_copy(rows_v.at[pl.ds(0, ch)], out_ref.at[pl.ds(off, ch)])

    return pl.kernel(
        body,
        out_type=jax.ShapeDtypeStruct((E0, d), dt),
        mesh=_mesh(),
        scratch_types=[pltpu.VMEM((CH0,), jnp.int32),
                       pltpu.VMEM((CH1,), jnp.int32),
                       pltpu.VMEM((CH0, d), dt),
                       pltpu.SemaphoreType.DMA],
        compiler_params=_sc_params(),
    )(table, idx)


def _sc_scatter(msg, dstidx, zrows, d):
    def body(msg_ref, dst_ref, z_ref, out_ref, idx_v0, idx_v1, rows_v, acc):
        c = lax.axis_index("c")
        s = lax.axis_index("s")
        wid = s * NC + c
        r0 = s * RPS
        pltpu.sync_copy(z_ref.at[pl.ds(r0, RPS)], acc.at[pl.ds(r0, RPS)])
        plsc.subcore_barrier()
        base = wid * EPW
        for off, ch, idx_v in ((base, CH0, idx_v0), (base + CH0, CH1, idx_v1)):
            pltpu.sync_copy(dst_ref.at[pl.ds(off, ch)], idx_v)
            pltpu.sync_copy(msg_ref.at[pl.ds(off, ch)], rows_v.at[pl.ds(0, ch)])
            pltpu.sync_copy(rows_v.at[pl.ds(0, ch)], acc.at[idx_v], add=True)
        plsc.subcore_barrier()
        pltpu.sync_copy(acc.at[pl.ds(r0, RPS)], out_ref.at[c, pl.ds(r0, RPS)])

    return pl.kernel(
        body,
        out_type=jax.ShapeDtypeStruct((NC, NP, d), jnp.float32),
        mesh=_mesh(),
        scratch_types=[pltpu.VMEM((CH0,), jnp.int32),
                       pltpu.VMEM((CH1,), jnp.int32),
                       pltpu.VMEM((CH0, d), jnp.float32),
                       pltpu.VMEM_SHARED((NP, d), jnp.float32)],
        compiler_params=_sc_params(),
    )(msg, dstidx, zrows)


def _sc_gcn(g, srcidx, dstidx, zrows, d):
    def body(g_ref, src_ref, dst_ref, z_ref, out_ref,
             si_v0, si_v1, di_v0, di_v1, rows_v, acc, sem):
        c = lax.axis_index("c")
        s = lax.axis_index("s")
        wid = s * NC + c
        r0 = s * RPS
        pltpu.sync_copy(z_ref.at[pl.ds(r0, RPS)], acc.at[pl.ds(r0, RPS)])
        plsc.subcore_barrier()
        base = wid * EPW
        for off, ch, si_v, di_v in ((base, CH0, si_v0, di_v0),
                                    (base + CH0, CH1, si_v1, di_v1)):
            pltpu.sync_copy(src_ref.at[pl.ds(off, ch)], si_v)
            pltpu.sync_copy(dst_ref.at[pl.ds(off, ch)], di_v)
            pltpu.async_copy(g_ref.at[si_v], rows_v.at[pl.ds(0, ch)], sem).wait()
            pltpu.sync_copy(rows_v.at[pl.ds(0, ch)], acc.at[di_v], add=True)
        plsc.subcore_barrier()
        pltpu.sync_copy(acc.at[pl.ds(r0, RPS)], out_ref.at[c, pl.ds(r0, RPS)])

    return pl.kernel(
        body,
        out_type=jax.ShapeDtypeStruct((NC, NP, d), jnp.float32),
        mesh=_mesh(),
        scratch_types=[pltpu.VMEM((CH0,), jnp.int32),
                       pltpu.VMEM((CH1,), jnp.int32),
                       pltpu.VMEM((CH0,), jnp.int32),
                       pltpu.VMEM((CH1,), jnp.int32),
                       pltpu.VMEM((CH0, d), jnp.float32),
                       pltpu.VMEM_SHARED((NP, d), jnp.float32),
                       pltpu.SemaphoreType.DMA],
        compiler_params=_sc_params(),
    )(g, srcidx, dstidx, zrows)


def _tc_edge_msg(xjp, eap, ra, rb, w, bem, deg_col):
    def body(xjp_ref, eap_ref, ra_ref, rb_ref, w_ref, bem_ref, out_ref):
        xjb = xjp_ref[...].astype(jnp.bfloat16)
        eab = eap_ref[...].astype(jnp.bfloat16)
        a = jnp.dot(xjb, ra_ref[...],
                    preferred_element_type=jnp.float32).astype(jnp.bfloat16)
        b = jnp.dot(eab, rb_ref[...],
                    preferred_element_type=jnp.float32).astype(jnp.bfloat16)
        m = jnp.dot(a * b, w_ref[...], preferred_element_type=jnp.float32)
        m = m + jnp.dot(xjb, bem_ref[...], preferred_element_type=jnp.float32)
        if deg_col is not None:
            lanes = lax.broadcasted_iota(jnp.int32, (PBS, 128), 1)
            m = m + jnp.where(lanes % 32 == deg_col, 1.0, 0.0)
        out_ref[...] = m

    return pl.pallas_call(
        body,
        grid=(EQ // PBS,),
        in_specs=[pl.BlockSpec((PBS, 128), lambda i: (i, 0)),
                  pl.BlockSpec((PBS, 64), lambda i: (i, 0)),
                  pl.BlockSpec((128, 2048), lambda i: (0, 0)),
                  pl.BlockSpec((64, 2048), lambda i: (0, 0)),
                  pl.BlockSpec((2048, 128), lambda i: (0, 0)),
                  pl.BlockSpec((128, 128), lambda i: (0, 0))],
        out_specs=pl.BlockSpec((PBS, 128), lambda i: (i, 0)),
        out_shape=jax.ShapeDtypeStruct((EQ, 128), jnp.float32),
    )(xjp, eap, ra, rb, w, bem)


def _tc_h1(a0, a1, xp, r1p, bi1):
    def body(a0_ref, a1_ref, x_ref, r_ref, b_ref, h_ref, dinv_ref):
        ssum = a0_ref[0] + a1_ref[0]
        hf = ssum + jnp.dot(x_ref[...], r_ref[...],
                            preferred_element_type=jnp.float32) + b_ref[...]
        lanes = lax.broadcasted_iota(jnp.int32, (NBS, 32), 1)
        h_ref[...] = jnp.where(lanes < 24, jnp.maximum(hf, 0.0), 0.0)
        deg = ssum[:, 24:25] + 1.0
        dinv_ref[...] = jnp.broadcast_to(lax.rsqrt(deg), (NBS, 8))

    return pl.pallas_call(
        body,
        grid=(NP // NBS,),
        in_specs=[pl.BlockSpec((1, NBS, 32), lambda i: (0, i, 0)),
                  pl.BlockSpec((1, NBS, 32), lambda i: (1, i, 0)),
                  pl.BlockSpec((NBS, 32), lambda i: (i, 0)),
                  pl.BlockSpec((32, 32), lambda i: (0, 0)),
                  pl.BlockSpec((1, 32), lambda i: (0, 0))],
        out_specs=[pl.BlockSpec((NBS, 32), lambda i: (i, 0)),
                   pl.BlockSpec((NBS, 8), lambda i: (i, 0))],
        out_shape=[jax.ShapeDtypeStruct((NP, 32), jnp.float32),
                   jax.ShapeDtypeStruct((NP, 8), jnp.float32)],
    )(a0, a1, xp, r1p, bi1)


def _tc_g(a0, a1, h1, dinv, r2p, bi2, wc):
    def body(a0_ref, a1_ref, h1_ref, dv_ref, r_ref, b_ref, wc_ref, g_ref):
        h2 = a0_ref[0] + a1_ref[0] + jnp.dot(
            h1_ref[...], r_ref[...], preferred_element_type=jnp.float32) + b_ref[...]
        h2 = jnp.maximum(h2, 0.0)
        g_ref[...] = dv_ref[:, 0:1] * jnp.dot(
            h2, wc_ref[...], preferred_element_type=jnp.float32)

    return pl.pallas_call(
        body,
        grid=(NP // NBS,),
        in_specs=[pl.BlockSpec((1, NBS, 32), lambda i: (0, i, 0)),
                  pl.BlockSpec((1, NBS, 32), lambda i: (1, i, 0)),
                  pl.BlockSpec((NBS, 32), lambda i: (i, 0)),
                  pl.BlockSpec((NBS, 8), lambda i: (i, 0)),
                  pl.BlockSpec((32, 32), lambda i: (0, 0)),
                  pl.BlockSpec((1, 32), lambda i: (0, 0)),
                  pl.BlockSpec((32, 16), lambda i: (0, 0))],
        out_specs=pl.BlockSpec((NBS, 16), lambda i: (i, 0)),
        out_shape=jax.ShapeDtypeStruct((NP, 16), jnp.float32),
    )(a0, a1, h1, dinv, r2p, bi2, wc)


def _tc_final(s0, s1, g, dinv, bc):
    def body(s0_ref, s1_ref, g_ref, dv_ref, b_ref, o_ref):
        o_ref[...] = dv_ref[:, 0:1] * (s0_ref[0] + s1_ref[0] + g_ref[...]) + b_ref[...]

    return pl.pallas_call(
        body,
        grid=(NP // NBS,),
        in_specs=[pl.BlockSpec((1, NBS, 16), lambda i: (0, i, 0)),
                  pl.BlockSpec((1, NBS, 16), lambda i: (1, i, 0)),
                  pl.BlockSpec((NBS, 16), lambda i: (i, 0)),
                  pl.BlockSpec((NBS, 8), lambda i: (i, 0)),
                  pl.BlockSpec((1, 16), lambda i: (0, 0))],
        out_specs=pl.BlockSpec((NBS, 16), lambda i: (i, 0)),
        out_shape=jax.ShapeDtypeStruct((NP, 16), jnp.float32),
    )(s0, s1, g, dinv, bc)


def kernel(x, edge_index, edge_attr, We1, be1, root1, bias1,
           We2, be2, root2, bias2, Wmu, bmu, Wls, bls):
    f32 = jnp.float32
    src = edge_index[0]
    dst = edge_index[1]
    ea = edge_attr
    xp = jnp.zeros((NP, 32), f32).at[:N0].set(x)

    bf16 = jnp.bfloat16
    colr = jnp.arange(2048)
    s_c, rem = colr // 512, colr % 512
    i_c, f_c = rem // 16, rem % 16
    rowa = jnp.arange(128)
    ra = ((rowa[:, None] // 32 == s_c[None, :])
          & (rowa[:, None] % 32 == i_c[None, :])).astype(bf16)
    rowb = jnp.arange(64)
    rb = ((rowb[:, None] // 16 == s_c[None, :])
          & (rowb[:, None] % 16 == f_c[None, :])).astype(bf16)

    eye4 = jnp.eye(4, dtype=f32)
    w1 = jnp.kron(eye4, jnp.zeros((512, 32), f32).at[:, :24].set(
        We1.reshape(16, 32, 24).transpose(1, 0, 2).reshape(512, 24))).astype(bf16)
    b1m = jnp.kron(eye4, jnp.zeros((32, 32), f32).at[:, :24].set(
        be1.reshape(32, 24))).astype(bf16)
    w2 = jnp.kron(eye4, jnp.zeros((512, 32), f32).at[:384, :16].set(
        We2.reshape(16, 24, 16).transpose(1, 0, 2).reshape(384, 16))).astype(bf16)
    b2m = jnp.kron(eye4, jnp.zeros((32, 32), f32).at[:24, :16].set(
        be2.reshape(24, 16))).astype(bf16)
    r1p = jnp.zeros((32, 32), f32).at[:, :24].set(root1)
    bi1 = jnp.zeros((1, 32), f32).at[0, :24].set(bias1)
    r2p = jnp.zeros((32, 32), f32).at[:24, :16].set(root2)
    bi2 = jnp.zeros((1, 32), f32).at[0, :16].set(bias2)
    wc = jnp.zeros((32, 16), f32).at[:16].set(jnp.concatenate([Wmu, Wls], axis=1))
    bc = jnp.concatenate([bmu, bls]).reshape(1, 16)
    z32 = jnp.zeros((NP, 32), f32)
    z16 = jnp.zeros((NP, 16), f32)

    eap = ea.reshape(EQ, 64)

    xj = _sc_gather(xp, src, 32, f32)
    msg1p = _tc_edge_msg(xj.reshape(EQ, 128), eap, ra, rb, w1, b1m, deg_col=24)
    acc1 = _sc_scatter(msg1p.reshape(E0, 32), dst, z32, 32)
    h1, dinv = _tc_h1(acc1, acc1, xp, r1p, bi1)

    h1j = _sc_gather(h1, src, 32, f32)
    msg2p = _tc_edge_msg(h1j.reshape(EQ, 128), eap, ra, rb, w2, b2m, deg_col=None)
    acc2 = _sc_scatter(msg2p.reshape(E0, 32), dst, z32, 32)
    g = _tc_g(acc2, acc2, h1, dinv, r2p, bi2, wc)

    s = _sc_gcn(g, src, dst, z16, 16)
    o = _tc_final(s, s, g, dinv, bc)
    return (o[:N0, :8], o[:N0, 8:16])
